# baseline (speedup 1.0000x reference)
import jax
import jax.numpy as jnp
from jax import lax
from jax.experimental import pallas as pl
from jax.experimental.pallas import tpu as pltpu
from jax.experimental.pallas import tpu_sc as plsc

B = 1024
S = 200
N = B * S
EMB = 64
POS_DIM = 32
OUT_D = EMB + POS_DIM
PAD_D = 128

NW = 32
NT = N // NW
G = 128
C = 256
GPB = C // G
NB = NT // C


def _emb_body(widx_hbm, pidx_hbm, wtab_hbm, ptab_hbm, out_hbm,
              idx_w, idx_p, comb_v, sem_i, sem_w, sem_p, sem_o):
    wid = lax.axis_index("s") * 2 + lax.axis_index("c")
    base_t = wid * NT

    def idx_copies(g, b):
        base = base_t + g * C
        return [
            pltpu.make_async_copy(widx_hbm.at[pl.ds(base, C)],
                                  idx_w.at[b], sem_i),
            pltpu.make_async_copy(pidx_hbm.at[pl.ds(base, C)],
                                  idx_p.at[b], sem_i),
        ]

    def word_gathers(b):
        return [
            pltpu.make_async_copy(
                wtab_hbm.at[idx_w.at[b, pl.ds(j * G, G)]],
                comb_v.at[b, pl.ds(j * G, G)], sem_w)
            for j in range(GPB)
        ]

    def pos_gathers(b):
        return [
            pltpu.async_copy(
                ptab_hbm.at[idx_p.at[b, pl.ds(j * G, G)]],
                comb_v.at[b, pl.ds(j * G, G)], sem_p, add=True)
            for j in range(GPB)
        ]

    def out_copies(g, b):
        base = base_t + g * C
        return [
            pltpu.make_async_copy(
                comb_v.at[b], out_hbm.at[pl.ds(base, C)], sem_o),
        ]

    idx_d = [idx_copies(g, g % 2) for g in range(NB)]
    out_d = [out_copies(g, g % 2) for g in range(NB)]

    for c in idx_d[0] + idx_d[1]:
        c.start()
    for g in range(NB):
        b = g % 2
        for c in idx_d[g]:
            c.wait()
        if g >= 2:
            for c in out_d[g - 2]:
                c.wait()
        wg = word_gathers(b)
        for c in wg:
            c.start()
        for c in wg:
            c.wait()
        pg = pos_gathers(b)
        for c in pg:
            c.wait()
        for c in out_d[g]:
            c.start()
        if g + 2 < NB:
            for c in idx_d[g + 2]:
                c.start()
    for g in (NB - 2, NB - 1):
        for c in out_d[g]:
            c.wait()


@jax.jit
def _emb_call(widx, pidx, wtab_p, ptab_s):
    mesh = plsc.VectorSubcoreMesh(core_axis_name="c", subcore_axis_name="s")
    f = pl.kernel(
        _emb_body,
        out_type=jax.ShapeDtypeStruct((N, PAD_D), jnp.float32),
        mesh=mesh,
        scratch_types=[
            pltpu.VMEM((2, C), jnp.int32),
            pltpu.VMEM((2, C), jnp.int32),
            pltpu.VMEM((2, C, PAD_D), jnp.float32),
            pltpu.SemaphoreType.DMA,
            pltpu.SemaphoreType.DMA,
            pltpu.SemaphoreType.DMA,
            pltpu.SemaphoreType.DMA,
        ],
    )
    return f(widx, pidx, wtab_p, ptab_s)


def kernel(word_inputs, pos_inputs, word_table, pos_table):
    widx = word_inputs.astype(jnp.int32).reshape(N)
    pidx = pos_inputs.astype(jnp.int32).reshape(N)
    wtab_p = jnp.pad(word_table, ((0, 0), (0, PAD_D - EMB)))
    ptab_s = jnp.pad(pos_table, ((0, 0), (EMB, PAD_D - EMB - POS_DIM)))
    out = _emb_call(widx, pidx, wtab_p, ptab_s)
    return out[:, :OUT_D].reshape(B, S, OUT_D)
